# Initial kernel scaffold; baseline (speedup 1.0000x reference)
import jax
import jax.numpy as jnp
from jax.experimental import pallas as pl


def kernel(inputs):
    raise NotImplementedError("write your pallas kernel here")


# speedup vs baseline: 124.9102x; 124.9102x over previous
import jax
import jax.numpy as jnp
from jax.experimental import pallas as pl
from jax.experimental.pallas import tpu as pltpu

G = 8
NBINS = 256
CLIP = 5.0


def _hist_lut_kernel(img_ref, perm_ref, tri_ref, lut_ref, gray_ref, *, th, tw, w):
    area = th * tw
    clip_lim = float(max(int(CLIP * area / NBINS), 1))
    x = img_ref[0]
    x = jnp.floor(jnp.clip(x, 0.0, 255.0))
    chans = jnp.dot(x.astype(jnp.bfloat16), perm_ref[...],
                    preferred_element_type=jnp.float32)
    gray = jnp.clip(
        jnp.round(0.114 * chans[:, :w] + 0.587 * chans[:, w:2 * w]
                  + 0.299 * chans[:, 2 * w:]),
        0.0, 255.0)
    gray_ref[0] = gray
    bins = jax.lax.broadcasted_iota(jnp.int32, (th, tw, NBINS), 2).astype(jnp.float32)
    luts = []
    for tx in range(G):
        v = gray[:, tx * tw:(tx + 1) * tw]
        onehot = (v[:, :, None] == bins).astype(jnp.float32)
        hist = jnp.sum(onehot, axis=(0, 1))[None, :]
        excess = jnp.sum(jnp.maximum(hist - clip_lim, 0.0))
        histc = jnp.minimum(hist, clip_lim) + excess * (1.0 / NBINS)
        cdf = jnp.dot(histc, tri_ref[...], preferred_element_type=jnp.float32)
        luts.append(jnp.clip(jnp.round(cdf * (255.0 / area)), 0.0, 255.0))
    lut_ref[0, 0] = jnp.concatenate(luts, axis=0)


def _interp_kernel(gray_ref, lut1_ref, lut2_ref, out_ref, *, th, tw, sh, w):
    s = pl.program_id(1)
    gray = gray_ref[0]
    l1 = lut1_ref[0, 0]
    l2 = lut2_ref[0, 0]
    rows = jax.lax.broadcasted_iota(jnp.int32, (sh, 1), 0).astype(jnp.float32)
    yf = (s.astype(jnp.float32) * float(sh) + rows) * (1.0 / th) - 0.5
    ya = yf - jnp.floor(yf)
    sw = tw // 2
    nseg = w // sw
    segs = []
    for c in range(nseg):
        x1 = min(max((c - 1) // 2, 0), G - 1)
        x2 = min((c - 1) // 2 + 1, G - 1)
        cols = jax.lax.broadcasted_iota(jnp.int32, (1, sw), 1).astype(jnp.float32) + float(c * sw)
        xf = cols * (1.0 / tw) - 0.5
        xa = xf - jnp.floor(xf)
        e = (1.0 - ya) * l1[x1][None, :] + ya * l2[x1][None, :]
        f = (1.0 - ya) * l1[x2][None, :] + ya * l2[x2][None, :]
        v = gray[:, c * sw:(c + 1) * sw]
        bins = jax.lax.broadcasted_iota(jnp.int32, (sh, sw, NBINS), 2).astype(jnp.float32)
        onehot = (v[:, :, None] == bins).astype(jnp.float32)
        re = jnp.sum(onehot * e[:, None, :], axis=2)
        rf = jnp.sum(onehot * f[:, None, :], axis=2)
        segs.append(re * (1.0 - xa) + rf * xa)
    res = jnp.clip(jnp.round(jnp.concatenate(segs, axis=1)), 0.0, 255.0)
    out_ref[0] = jnp.repeat(res, 3, axis=1)


def kernel(inputs):
    img = inputs
    B, H, W, _ = img.shape
    th, tw = H // G, W // G
    import functools

    tri = (jnp.arange(NBINS)[:, None] <= jnp.arange(NBINS)[None, :]).astype(jnp.float32)
    img = img.reshape(B, H, W * 3)
    j = jnp.arange(W * 3)
    perm = (jnp.arange(W * 3)[None, :] == (W * (j % 3) + j // 3)[:, None]
            ).astype(jnp.bfloat16)

    luts, gray = pl.pallas_call(
        functools.partial(_hist_lut_kernel, th=th, tw=tw, w=W),
        grid=(B, G),
        in_specs=[
            pl.BlockSpec((1, th, W * 3), lambda b, ty: (b, ty, 0)),
            pl.BlockSpec((W * 3, W * 3), lambda b, ty: (0, 0)),
            pl.BlockSpec((NBINS, NBINS), lambda b, ty: (0, 0)),
        ],
        out_specs=[
            pl.BlockSpec((1, 1, G, NBINS), lambda b, ty: (b, ty, 0, 0)),
            pl.BlockSpec((1, th, W), lambda b, ty: (b, ty, 0)),
        ],
        out_shape=[
            jax.ShapeDtypeStruct((B, G, G, NBINS), jnp.float32),
            jax.ShapeDtypeStruct((B, H, W), jnp.float32),
        ],
        compiler_params=pltpu.CompilerParams(
            dimension_semantics=("parallel", "arbitrary")),
    )(img, perm, tri)

    sh = th // 2
    nstripe = H // sh

    def y1_map(b, s):
        return (b, jnp.clip((s - 1) // 2, 0, G - 1), 0, 0)

    def y2_map(b, s):
        return (b, jnp.clip((s - 1) // 2 + 1, 0, G - 1), 0, 0)

    out = pl.pallas_call(
        functools.partial(_interp_kernel, th=th, tw=tw, sh=sh, w=W),
        grid=(B, nstripe),
        in_specs=[
            pl.BlockSpec((1, sh, W), lambda b, s: (b, s, 0)),
            pl.BlockSpec((1, 1, G, NBINS), y1_map),
            pl.BlockSpec((1, 1, G, NBINS), y2_map),
        ],
        out_specs=pl.BlockSpec((1, sh, W * 3), lambda b, s: (b, s, 0)),
        out_shape=jax.ShapeDtypeStruct((B, H, W * 3), jnp.float32),
        compiler_params=pltpu.CompilerParams(
            dimension_semantics=("parallel", "arbitrary")),
    )(gray, luts, luts)
    return out.reshape(B, H, W, 3)
